# baseline (speedup 1.0000x reference)
import functools

import jax
import jax.numpy as jnp
from jax import lax
from jax.experimental import pallas as pl
from jax.experimental.pallas import tpu as pltpu
from jax.experimental.pallas import tpu_sc as plsc

EDGE_BLOCK = 512
SC_CORES = 2
SC_SUBCORES = 16
SC_WORKERS = SC_CORES * SC_SUBCORES
SC_CHUNK = 64


def _sc_mesh():
    return plsc.VectorSubcoreMesh(core_axis_name="c", subcore_axis_name="s",
                                  num_cores=SC_CORES, num_subcores=SC_SUBCORES)


def _gather_add_body(xa_hbm, xb_hbm, ea_hbm, ids_hbm, dstst_hbm, srcst_hbm,
                     g_hbm, eas_hbm, idv, dstv, srcv, ra, rb, re_, sema, semb,
                     seme, *, spa, c, f, n, e, de):
    wid = lax.axis_index("s") * SC_CORES + lax.axis_index("c")
    nch = spa // c

    def chunk(t, _):
        base = wid * spa + t * c
        pltpu.sync_copy(ids_hbm.at[pl.ds(base, c)], idv)
        pltpu.sync_copy(dstst_hbm.at[pl.ds(base, c)], dstv)
        pltpu.sync_copy(srcst_hbm.at[pl.ds(base, c)], srcv)
        for s in range(c // 16):
            sl = pl.ds(s * 16, 16)
            idv[sl] = jnp.clip(idv[sl], 0, e - 1)
            dstv[sl] = jnp.clip(dstv[sl], 0, n - 1)
            srcv[sl] = jnp.clip(srcv[sl], 0, n - 1)
        cpa = pltpu.async_copy(xa_hbm.at[dstv], ra, sema)
        cpb = pltpu.async_copy(xb_hbm.at[srcv], rb, semb)
        cpe = pltpu.async_copy(ea_hbm.at[idv], re_, seme)
        cpa.wait()
        cpb.wait()
        cpe.wait()

        def add_row(i, _):
            for k in range(f // 16):
                sl = pl.ds(k * 16, 16)
                ra[i, sl] = ra[i, sl] + rb[i, sl]
            return 0

        lax.fori_loop(0, c, add_row, 0)
        pltpu.sync_copy(ra, g_hbm.at[pl.ds(base, c)])
        pltpu.sync_copy(re_, eas_hbm.at[pl.ds(base, c)])
        return 0

    lax.fori_loop(0, nch, chunk, 0)


def _sc_gather_add(xa, xb, ea, staged):
    ids, dsts, srcs, cnt, lp = staged
    n, f = xa.shape
    e, de = ea.shape
    tot = ids.shape[0]
    spa = tot // SC_WORKERS
    c = SC_CHUNK
    assert spa % c == 0
    kfn = pl.kernel(
        functools.partial(_gather_add_body, spa=spa, c=c, f=f, n=n, e=e, de=de),
        out_type=(
            jax.ShapeDtypeStruct((tot, f), jnp.float32),
            jax.ShapeDtypeStruct((tot, de), jnp.float32),
        ),
        mesh=_sc_mesh(),
        compiler_params=pltpu.CompilerParams(use_tc_tiling_on_sc=False,
                                             needs_layout_passes=False),
        scratch_types=[
            pltpu.VMEM((c,), jnp.int32),
            pltpu.VMEM((c,), jnp.int32),
            pltpu.VMEM((c,), jnp.int32),
            pltpu.VMEM((c, f), jnp.float32),
            pltpu.VMEM((c, f), jnp.float32),
            pltpu.VMEM((c, de), jnp.float32),
            pltpu.SemaphoreType.DMA,
            pltpu.SemaphoreType.DMA,
            pltpu.SemaphoreType.DMA,
        ],
    )
    return kfn(xa, xb, ea, ids, dsts, srcs)


def _lrelu(x):
    return jnp.where(x > 0, x, 0.01 * x)


def _pre_body(x_ref, wd_ref, ws_ref, xa_ref, xb_ref):
    x = x_ref[...]
    xa_ref[...] = jnp.dot(x, wd_ref[...], preferred_element_type=jnp.float32)
    xb_ref[...] = jnp.dot(x, ws_ref[...], preferred_element_type=jnp.float32)


def _node_pre(x, wdT, wsT):
    n, f = x.shape
    fo = wdT.shape[1]
    return pl.pallas_call(
        _pre_body,
        out_shape=(
            jax.ShapeDtypeStruct((n, fo), jnp.float32),
            jax.ShapeDtypeStruct((n, fo), jnp.float32),
        ),
    )(x, wdT, wsT)


def _edge_body(g_ref, ea_ref, weT_ref, mb1_ref, mw2T_ref, mb2_ref, msg_ref):
    hidden = (g_ref[...]
              + jnp.dot(ea_ref[...], weT_ref[...], preferred_element_type=jnp.float32)
              + mb1_ref[...])
    hidden = _lrelu(hidden)
    msg_ref[...] = jnp.dot(hidden, mw2T_ref[...], preferred_element_type=jnp.float32) + mb2_ref[...]


def _edge_dense(g, ea, weT, mb1, mw2T, mb2):
    f = g.shape[1]
    fo = mw2T.shape[1]
    e = g.shape[0]
    b = EDGE_BLOCK
    assert e % b == 0, (e, b)
    grid = e // b
    return pl.pallas_call(
        _edge_body,
        grid=(grid,),
        in_specs=[
            pl.BlockSpec((b, f), lambda i: (i, 0)),
            pl.BlockSpec((b, ea.shape[1]), lambda i: (i, 0)),
            pl.BlockSpec(weT.shape, lambda i: (0, 0)),
            pl.BlockSpec(mb1.shape, lambda i: (0, 0)),
            pl.BlockSpec(mw2T.shape, lambda i: (0, 0)),
            pl.BlockSpec(mb2.shape, lambda i: (0, 0)),
        ],
        out_specs=pl.BlockSpec((b, fo), lambda i: (i, 0)),
        out_shape=jax.ShapeDtypeStruct((e, fo), jnp.float32),
    )(g, ea, weT, mb1, mw2T, mb2)


BUCKET_SHIFT = 6
BUCKET_NODES = 1 << BUCKET_SHIFT
SCAT_CHUNK = 64


def _iota16():
    return lax.broadcasted_iota(jnp.int32, (16,), 0)


def _ext_i32(ref, idx):
    base = (idx >> 4) << 4
    v = ref[pl.ds(base, 16)]
    m = _iota16() == (idx & 15)
    return jnp.max(jnp.where(m, v, jnp.int32(-(2 ** 31))))


def _put_i32(ref, idx, val):
    base = (idx >> 4) << 4
    sl = pl.ds(base, 16)
    v = ref[sl]
    m = _iota16() == (idx & 15)
    ref[sl] = jnp.where(m, val, v)


def _inc_i32(ref, idx):
    base = (idx >> 4) << 4
    sl = pl.ds(base, 16)
    v = ref[sl]
    m = _iota16() == (idx & 15)
    ref[sl] = v + jnp.where(m, 1, 0)


def _bucket_body(dst_hbm, src_hbm, ids_hbm, dsts_hbm, srcs_hbm, cnt_hbm,
                 lp_hbm, dstv, srcv, ids_st, dst_st, src_st, cnt, cur, lpl,
                 *, ew, nb, nb_pad, spa):
    wid = lax.axis_index("s") * SC_CORES + lax.axis_index("c")
    pltpu.sync_copy(dst_hbm.at[pl.ds(wid * ew, ew)], dstv)
    pltpu.sync_copy(src_hbm.at[pl.ds(wid * ew, ew)], srcv)

    zeros = jnp.zeros((16,), jnp.int32)
    for s in range(nb_pad // 16):
        cnt[pl.ds(s * 16, 16)] = zeros
        cur[pl.ds(s * 16, 16)] = zeros

    def hist(g, _):
        bv = dstv[pl.ds(g * 16, 16)] >> BUCKET_SHIFT
        for k in range(16):
            _inc_i32(cnt, bv[k])
        return 0

    lax.fori_loop(0, ew // 16, hist, 0)

    running = jnp.int32(0)
    for s in range(nb_pad // 16):
        sl = pl.ds(s * 16, 16)
        pvec = (cnt[sl] + 7) & -8
        cum = jnp.cumsum(pvec)
        lpl[sl] = cum - pvec + running
        running = running + cum[15]

    def place(g, _):
        dv = dstv[pl.ds(g * 16, 16)]
        sv = srcv[pl.ds(g * 16, 16)]
        bv = dv >> BUCKET_SHIFT
        for k in range(16):
            b = bv[k]
            pos = _ext_i32(lpl, b) + _ext_i32(cur, b)
            _put_i32(ids_st, pos, wid * ew + g * 16 + k)
            _put_i32(dst_st, pos, dv[k])
            _put_i32(src_st, pos, sv[k])
            _inc_i32(cur, b)
        return 0

    lax.fori_loop(0, ew // 16, place, 0)

    def pad(j, _):
        c = _ext_i32(cnt, j)
        p = (c + 7) & -8
        lpb = _ext_i32(lpl, j)

        @pl.when(c > 0)
        def _():
            fid = _ext_i32(ids_st, lpb)
            fd = _ext_i32(dst_st, lpb)
            fs = _ext_i32(src_st, lpb)

            def fill(t, _):
                _put_i32(ids_st, lpb + t, fid)
                _put_i32(dst_st, lpb + t, fd)
                _put_i32(src_st, lpb + t, fs)
                return 0

            lax.fori_loop(c, p, fill, 0)

        return 0

    lax.fori_loop(0, nb, pad, 0)

    pltpu.sync_copy(ids_st, ids_hbm.at[pl.ds(wid * spa, spa)])
    pltpu.sync_copy(dst_st, dsts_hbm.at[pl.ds(wid * spa, spa)])
    pltpu.sync_copy(src_st, srcs_hbm.at[pl.ds(wid * spa, spa)])
    pltpu.sync_copy(cnt, cnt_hbm.at[pl.ds(wid * nb_pad, nb_pad)])
    pltpu.sync_copy(lpl, lp_hbm.at[pl.ds(wid * nb_pad, nb_pad)])


def _sc_bucket(dst, src, n):
    e = dst.shape[0]
    assert e % SC_WORKERS == 0
    ew = e // SC_WORKERS
    nb = (n + BUCKET_NODES - 1) // BUCKET_NODES
    nb_pad = ((nb + 15) // 16) * 16
    spa = ew + nb * 8 + SCAT_CHUNK
    spa = ((spa + SCAT_CHUNK - 1) // SCAT_CHUNK) * SCAT_CHUNK
    kfn = pl.kernel(
        functools.partial(_bucket_body, ew=ew, nb=nb, nb_pad=nb_pad, spa=spa),
        out_type=(
            jax.ShapeDtypeStruct((SC_WORKERS * spa,), jnp.int32),
            jax.ShapeDtypeStruct((SC_WORKERS * spa,), jnp.int32),
            jax.ShapeDtypeStruct((SC_WORKERS * spa,), jnp.int32),
            jax.ShapeDtypeStruct((SC_WORKERS * nb_pad,), jnp.int32),
            jax.ShapeDtypeStruct((SC_WORKERS * nb_pad,), jnp.int32),
        ),
        mesh=_sc_mesh(),
        compiler_params=pltpu.CompilerParams(use_tc_tiling_on_sc=False,
                                             needs_layout_passes=False),
        scratch_types=[
            pltpu.VMEM((ew,), jnp.int32),
            pltpu.VMEM((ew,), jnp.int32),
            pltpu.VMEM((spa,), jnp.int32),
            pltpu.VMEM((spa,), jnp.int32),
            pltpu.VMEM((spa,), jnp.int32),
            pltpu.VMEM((nb_pad,), jnp.int32),
            pltpu.VMEM((nb_pad,), jnp.int32),
            pltpu.VMEM((nb_pad,), jnp.int32),
        ],
    )
    return kfn(dst, src)


def _scatmax_body(msg_hbm, dsts_hbm, lp_hbm, agg_hbm,
                  lp_v, ddv, rows, acc, *, f, nb, nb_pad, spa, rpw):
    wid = lax.axis_index("s") * SC_CORES + lax.axis_index("c")
    pltpu.sync_copy(lp_hbm, lp_v)
    ninf = jnp.full((16,), -jnp.inf, jnp.float32)
    nsl = f // 16
    acc_rows = rpw * BUCKET_NODES
    dump = acc_rows
    jlo = wid * rpw
    base_node = jlo * BUCKET_NODES

    def init_row(i, _):
        for k in range(nsl):
            acc[i, pl.ds(k * 16, 16)] = ninf
        return 0

    lax.fori_loop(0, acc_rows + 1, init_row, 0)

    def per_worker(w2, _):
        lo = _ext_i32(lp_v, w2 * nb_pad + jnp.minimum(jlo, nb_pad - 1))
        hi = _ext_i32(lp_v, w2 * nb_pad + jnp.minimum(jlo + rpw, nb_pad - 1))
        span = jnp.clip(hi - lo, 0, spa)
        lo = jnp.clip(lo, 0, spa - SCAT_CHUNK)
        nch = (span + SCAT_CHUNK - 1) >> 6

        def chunk(t, _):
            off = pl.multiple_of(
                w2 * spa + jnp.minimum(lo + t * SCAT_CHUNK, spa - SCAT_CHUNK), 8)
            pltpu.sync_copy(dsts_hbm.at[pl.ds(off, SCAT_CHUNK)], ddv)
            pltpu.sync_copy(msg_hbm.at[pl.ds(off, SCAT_CHUNK)], rows)

            for g in range(SCAT_CHUNK // 16):
                ldg = jnp.clip(ddv[pl.ds(g * 16, 16)] - base_node,
                               0, acc_rows - 1)
                for k in range(16):
                    i = g * 16 + k
                    ok = (t * SCAT_CHUNK + i) < span
                    ld = jnp.where(ok, ldg[k], dump)
                    for s in range(nsl):
                        sl = pl.ds(s * 16, 16)
                        acc[ld, sl] = jnp.maximum(acc[ld, sl], rows[i, sl])
            return 0

        lax.fori_loop(0, nch, chunk, 0)
        return 0

    lax.fori_loop(0, SC_WORKERS, per_worker, 0)
    pltpu.sync_copy(acc.at[pl.ds(0, acc_rows)],
                    agg_hbm.at[pl.ds(base_node, acc_rows)])


def _sc_segment_max(msg, staged, n):
    ids, dsts, srcs, cnt, lp = staged
    tot, f = msg.shape
    nb = (n + BUCKET_NODES - 1) // BUCKET_NODES
    nb_pad = ((nb + 15) // 16) * 16
    spa = tot // SC_WORKERS
    rpw = (nb + SC_WORKERS - 1) // SC_WORKERS
    kfn = pl.kernel(
        functools.partial(_scatmax_body, f=f, nb=nb, nb_pad=nb_pad,
                          spa=spa, rpw=rpw),
        out_type=jax.ShapeDtypeStruct((SC_WORKERS * rpw * BUCKET_NODES, f),
                                      jnp.float32),
        mesh=_sc_mesh(),
        compiler_params=pltpu.CompilerParams(use_tc_tiling_on_sc=False,
                                             needs_layout_passes=False),
        scratch_types=[
            pltpu.VMEM((SC_WORKERS * nb_pad,), jnp.int32),
            pltpu.VMEM((SCAT_CHUNK,), jnp.int32),
            pltpu.VMEM((SCAT_CHUNK, f), jnp.float32),
            pltpu.VMEM((rpw * BUCKET_NODES + 8, f), jnp.float32),
        ],
    )
    aggp = kfn(msg, dsts, lp)
    return aggp[:n]


def _update_body(x_ref, agg_ref, uxT_ref, uaT_ref, ub1_ref, uw2T_ref, ub2_ref, out_ref):
    agg = agg_ref[...]
    agg = jnp.where(jnp.isfinite(agg), agg, 0.0)
    hidden = (jnp.dot(x_ref[...], uxT_ref[...], preferred_element_type=jnp.float32)
              + jnp.dot(agg, uaT_ref[...], preferred_element_type=jnp.float32)
              + ub1_ref[...])
    hidden = _lrelu(hidden)
    h = jnp.dot(hidden, uw2T_ref[...], preferred_element_type=jnp.float32) + ub2_ref[...]
    out_ref[...] = jnp.maximum(h, 0.0)


def _node_update(x, agg, uxT, uaT, ub1, uw2T, ub2):
    n = x.shape[0]
    fo = uw2T.shape[1]
    return pl.pallas_call(
        _update_body,
        out_shape=jax.ShapeDtypeStruct((n, fo), jnp.float32),
    )(x, agg, uxT, uaT, ub1, uw2T, ub2)


def _final_body(h_ref, fwT_ref, fb_ref, out_ref):
    out_ref[...] = jnp.dot(h_ref[...], fwT_ref[...], preferred_element_type=jnp.float32) + fb_ref[...]


def _final(h, fwT, fb):
    n = h.shape[0]
    fo = fwT.shape[1]
    return pl.pallas_call(
        _final_body,
        out_shape=jax.ShapeDtypeStruct((n, fo), jnp.float32),
    )(h, fwT, fb)


def _mp_layer(x, ea, staged, mW1, mb1, mW2, mb2, uW1, ub1, uW2, ub2):
    f = x.shape[1]
    wdT = jnp.transpose(mW1[:, :f])
    wsT = jnp.transpose(mW1[:, f:2 * f])
    weT = jnp.transpose(mW1[:, 2 * f:])
    xa, xb = _node_pre(x, wdT, wsT)
    g, ea_st = _sc_gather_add(xa, xb, ea, staged)
    msg = _edge_dense(g, ea_st, weT, mb1[None, :], jnp.transpose(mW2), mb2[None, :])
    agg = _sc_segment_max(msg, staged, x.shape[0])
    uxT = jnp.transpose(uW1[:, :f])
    uaT = jnp.transpose(uW1[:, f:])
    return _node_update(x, agg, uxT, uaT, ub1[None, :], jnp.transpose(uW2), ub2[None, :])


def kernel(feature, edge_index, edge_attr, mp1_mW1, mp1_mb1, mp1_mW2, mp1_mb2,
           mp1_uW1, mp1_ub1, mp1_uW2, mp1_ub2, mp2_mW1, mp2_mb1, mp2_mW2, mp2_mb2,
           mp2_uW1, mp2_ub1, mp2_uW2, mp2_ub2, fW, fb):
    src = edge_index[0]
    dst = edge_index[1]
    staged = _sc_bucket(dst, src, feature.shape[0])
    h = _mp_layer(feature, edge_attr, staged,
                  mp1_mW1, mp1_mb1, mp1_mW2, mp1_mb2,
                  mp1_uW1, mp1_ub1, mp1_uW2, mp1_ub2)
    h = _mp_layer(h, edge_attr, staged,
                  mp2_mW1, mp2_mb1, mp2_mW2, mp2_mb2,
                  mp2_uW1, mp2_ub1, mp2_uW2, mp2_ub2)
    return _final(h, jnp.transpose(fW), fb[None, :])


# speedup vs baseline: 1.0405x; 1.0405x over previous
import functools

import jax
import jax.numpy as jnp
from jax import lax
from jax.experimental import pallas as pl
from jax.experimental.pallas import tpu as pltpu
from jax.experimental.pallas import tpu_sc as plsc

EDGE_BLOCK = 512
SC_CORES = 2
SC_SUBCORES = 16
SC_WORKERS = SC_CORES * SC_SUBCORES
SC_CHUNK = 64


def _sc_mesh():
    return plsc.VectorSubcoreMesh(core_axis_name="c", subcore_axis_name="s",
                                  num_cores=SC_CORES, num_subcores=SC_SUBCORES)


def _gather_add_body(xa_hbm, xb_hbm, dstst_hbm, srcst_hbm, g_hbm,
                     dv0, sv0, dv1, sv1, ra0, rb0, ra1, rb1,
                     sa0, sb0, sa1, sb1, *, spa, c, f, n):
    wid = lax.axis_index("s") * SC_CORES + lax.axis_index("c")
    nch = spa // c

    def load_issue(base, dv, sv, ra, rb, sa, sb):
        pltpu.sync_copy(dstst_hbm.at[pl.ds(base, c)], dv)
        pltpu.sync_copy(srcst_hbm.at[pl.ds(base, c)], sv)
        for s in range(c // 16):
            sl = pl.ds(s * 16, 16)
            dv[sl] = jnp.clip(dv[sl], 0, n - 1)
            sv[sl] = jnp.clip(sv[sl], 0, n - 1)
        cpa = pltpu.async_copy(xa_hbm.at[dv], ra, sa)
        cpb = pltpu.async_copy(xb_hbm.at[sv], rb, sb)
        return cpa, cpb

    def process_store(base, ra, rb, cps):
        cpa, cpb = cps
        cpa.wait()
        cpb.wait()

        def add_row(i, _):
            for k in range(f // 16):
                sl = pl.ds(k * 16, 16)
                ra[i, sl] = ra[i, sl] + rb[i, sl]
            return 0

        lax.fori_loop(0, c, add_row, 0)
        pltpu.sync_copy(ra, g_hbm.at[pl.ds(base, c)])

    def pair(s2, _):
        b0 = wid * spa + (2 * s2) * c
        b1 = b0 + c
        h0 = load_issue(b0, dv0, sv0, ra0, rb0, sa0, sb0)
        h1 = load_issue(b1, dv1, sv1, ra1, rb1, sa1, sb1)
        process_store(b0, ra0, rb0, h0)
        process_store(b1, ra1, rb1, h1)
        return 0

    lax.fori_loop(0, nch // 2, pair, 0)


def _sc_gather_add(xa, xb, staged):
    eas, dsts, srcs, cnt, lp = staged
    n, f = xa.shape
    tot = dsts.shape[0]
    spa = tot // SC_WORKERS
    c = SC_CHUNK
    assert spa % (2 * c) == 0
    kfn = pl.kernel(
        functools.partial(_gather_add_body, spa=spa, c=c, f=f, n=n),
        out_type=jax.ShapeDtypeStruct((tot, f), jnp.float32),
        mesh=_sc_mesh(),
        compiler_params=pltpu.CompilerParams(use_tc_tiling_on_sc=False,
                                             needs_layout_passes=False),
        scratch_types=[
            pltpu.VMEM((c,), jnp.int32),
            pltpu.VMEM((c,), jnp.int32),
            pltpu.VMEM((c,), jnp.int32),
            pltpu.VMEM((c,), jnp.int32),
            pltpu.VMEM((c, f), jnp.float32),
            pltpu.VMEM((c, f), jnp.float32),
            pltpu.VMEM((c, f), jnp.float32),
            pltpu.VMEM((c, f), jnp.float32),
            pltpu.SemaphoreType.DMA,
            pltpu.SemaphoreType.DMA,
            pltpu.SemaphoreType.DMA,
            pltpu.SemaphoreType.DMA,
        ],
    )
    return kfn(xa, xb, dsts, srcs)


def _lrelu(x):
    return jnp.where(x > 0, x, 0.01 * x)


def _pre_body(x_ref, wd_ref, ws_ref, xa_ref, xb_ref):
    x = x_ref[...]
    xa_ref[...] = jnp.dot(x, wd_ref[...], preferred_element_type=jnp.float32)
    xb_ref[...] = jnp.dot(x, ws_ref[...], preferred_element_type=jnp.float32)


def _node_pre(x, wdT, wsT):
    n, f = x.shape
    fo = wdT.shape[1]
    return pl.pallas_call(
        _pre_body,
        out_shape=(
            jax.ShapeDtypeStruct((n, fo), jnp.float32),
            jax.ShapeDtypeStruct((n, fo), jnp.float32),
        ),
    )(x, wdT, wsT)


def _edge_body(g_ref, ea_ref, weT_ref, mb1_ref, mw2T_ref, mb2_ref, msg_ref):
    hidden = (g_ref[...]
              + jnp.dot(ea_ref[...], weT_ref[...], preferred_element_type=jnp.float32)
              + mb1_ref[...])
    hidden = _lrelu(hidden)
    msg_ref[...] = jnp.dot(hidden, mw2T_ref[...], preferred_element_type=jnp.float32) + mb2_ref[...]


def _edge_dense(g, ea, weT, mb1, mw2T, mb2):
    f = g.shape[1]
    fo = mw2T.shape[1]
    e = g.shape[0]
    b = EDGE_BLOCK
    assert e % b == 0, (e, b)
    grid = e // b
    return pl.pallas_call(
        _edge_body,
        grid=(grid,),
        in_specs=[
            pl.BlockSpec((b, f), lambda i: (i, 0)),
            pl.BlockSpec((b, ea.shape[1]), lambda i: (i, 0)),
            pl.BlockSpec(weT.shape, lambda i: (0, 0)),
            pl.BlockSpec(mb1.shape, lambda i: (0, 0)),
            pl.BlockSpec(mw2T.shape, lambda i: (0, 0)),
            pl.BlockSpec(mb2.shape, lambda i: (0, 0)),
        ],
        out_specs=pl.BlockSpec((b, fo), lambda i: (i, 0)),
        out_shape=jax.ShapeDtypeStruct((e, fo), jnp.float32),
    )(g, ea, weT, mb1, mw2T, mb2)


BUCKET_SHIFT = 6
BUCKET_NODES = 1 << BUCKET_SHIFT
SCAT_CHUNK = 64


def _iota16():
    return lax.broadcasted_iota(jnp.int32, (16,), 0)


def _ext_i32(ref, idx):
    base = (idx >> 4) << 4
    v = ref[pl.ds(base, 16)]
    m = _iota16() == (idx & 15)
    return jnp.max(jnp.where(m, v, jnp.int32(-(2 ** 31))))


def _put_i32(ref, idx, val):
    base = (idx >> 4) << 4
    sl = pl.ds(base, 16)
    v = ref[sl]
    m = _iota16() == (idx & 15)
    ref[sl] = jnp.where(m, val, v)


def _inc_i32(ref, idx):
    base = (idx >> 4) << 4
    sl = pl.ds(base, 16)
    v = ref[sl]
    m = _iota16() == (idx & 15)
    ref[sl] = v + jnp.where(m, 1, 0)


def _bucket_body(dst_hbm, src_hbm, ea_hbm, eas_hbm, dsts_hbm, srcs_hbm,
                 cnt_hbm, lp_hbm, dstv, srcv, ids_st, dst_st, src_st, cnt,
                 cur, lpl, idxg, eav, semg, *, ew, nb, nb_pad, spa, e, de):
    wid = lax.axis_index("s") * SC_CORES + lax.axis_index("c")
    pltpu.sync_copy(dst_hbm.at[pl.ds(wid * ew, ew)], dstv)
    pltpu.sync_copy(src_hbm.at[pl.ds(wid * ew, ew)], srcv)

    zeros = jnp.zeros((16,), jnp.int32)
    for s in range(nb_pad // 16):
        cnt[pl.ds(s * 16, 16)] = zeros
        cur[pl.ds(s * 16, 16)] = zeros

    def hist(g, _):
        bv = dstv[pl.ds(g * 16, 16)] >> BUCKET_SHIFT
        for k in range(16):
            _inc_i32(cnt, bv[k])
        return 0

    lax.fori_loop(0, ew // 16, hist, 0)

    running = jnp.int32(0)
    for s in range(nb_pad // 16):
        sl = pl.ds(s * 16, 16)
        pvec = (cnt[sl] + 7) & -8
        cum = jnp.cumsum(pvec)
        lpl[sl] = cum - pvec + running
        running = running + cum[15]

    def place(g, _):
        dv = dstv[pl.ds(g * 16, 16)]
        sv = srcv[pl.ds(g * 16, 16)]
        bv = dv >> BUCKET_SHIFT
        for k in range(16):
            b = bv[k]
            pos = _ext_i32(lpl, b) + _ext_i32(cur, b)
            _put_i32(ids_st, pos, wid * ew + g * 16 + k)
            _put_i32(dst_st, pos, dv[k])
            _put_i32(src_st, pos, sv[k])
            _inc_i32(cur, b)
        return 0

    lax.fori_loop(0, ew // 16, place, 0)

    def pad(j, _):
        c = _ext_i32(cnt, j)
        p = (c + 7) & -8
        lpb = _ext_i32(lpl, j)

        @pl.when(c > 0)
        def _():
            fid = _ext_i32(ids_st, lpb)
            fd = _ext_i32(dst_st, lpb)
            fs = _ext_i32(src_st, lpb)

            def fill(t, _):
                _put_i32(ids_st, lpb + t, fid)
                _put_i32(dst_st, lpb + t, fd)
                _put_i32(src_st, lpb + t, fs)
                return 0

            lax.fori_loop(c, p, fill, 0)

        return 0

    lax.fori_loop(0, nb, pad, 0)

    def egather(t, _):
        base = t * SCAT_CHUNK
        for s2 in range(SCAT_CHUNK // 16):
            sl = pl.ds(s2 * 16, 16)
            idxg[sl] = jnp.clip(ids_st[pl.ds(base + s2 * 16, 16)], 0, e - 1)
        cp = pltpu.async_copy(ea_hbm.at[idxg], eav, semg)
        cp.wait()
        pltpu.sync_copy(eav, eas_hbm.at[pl.ds(wid * spa + base, SCAT_CHUNK)])
        return 0

    lax.fori_loop(0, spa // SCAT_CHUNK, egather, 0)

    pltpu.sync_copy(dst_st, dsts_hbm.at[pl.ds(wid * spa, spa)])
    pltpu.sync_copy(src_st, srcs_hbm.at[pl.ds(wid * spa, spa)])
    pltpu.sync_copy(cnt, cnt_hbm.at[pl.ds(wid * nb_pad, nb_pad)])
    pltpu.sync_copy(lpl, lp_hbm.at[pl.ds(wid * nb_pad, nb_pad)])


def _sc_bucket(dst, src, ea, n):
    e = dst.shape[0]
    de = ea.shape[1]
    assert e % SC_WORKERS == 0
    ew = e // SC_WORKERS
    nb = (n + BUCKET_NODES - 1) // BUCKET_NODES
    nb_pad = ((nb + 15) // 16) * 16
    spa = ew + nb * 8 + SCAT_CHUNK
    spa = ((spa + 2 * SCAT_CHUNK - 1) // (2 * SCAT_CHUNK)) * (2 * SCAT_CHUNK)
    kfn = pl.kernel(
        functools.partial(_bucket_body, ew=ew, nb=nb, nb_pad=nb_pad, spa=spa,
                          e=e, de=de),
        out_type=(
            jax.ShapeDtypeStruct((SC_WORKERS * spa, de), jnp.float32),
            jax.ShapeDtypeStruct((SC_WORKERS * spa,), jnp.int32),
            jax.ShapeDtypeStruct((SC_WORKERS * spa,), jnp.int32),
            jax.ShapeDtypeStruct((SC_WORKERS * nb_pad,), jnp.int32),
            jax.ShapeDtypeStruct((SC_WORKERS * nb_pad,), jnp.int32),
        ),
        mesh=_sc_mesh(),
        compiler_params=pltpu.CompilerParams(use_tc_tiling_on_sc=False,
                                             needs_layout_passes=False),
        scratch_types=[
            pltpu.VMEM((ew,), jnp.int32),
            pltpu.VMEM((ew,), jnp.int32),
            pltpu.VMEM((spa,), jnp.int32),
            pltpu.VMEM((spa,), jnp.int32),
            pltpu.VMEM((spa,), jnp.int32),
            pltpu.VMEM((nb_pad,), jnp.int32),
            pltpu.VMEM((nb_pad,), jnp.int32),
            pltpu.VMEM((nb_pad,), jnp.int32),
            pltpu.VMEM((SCAT_CHUNK,), jnp.int32),
            pltpu.VMEM((SCAT_CHUNK, de), jnp.float32),
            pltpu.SemaphoreType.DMA,
        ],
    )
    return kfn(dst, src, ea)


def _scatmax_body(msg_hbm, dsts_hbm, lp_hbm, agg_hbm,
                  lp_v, ddv, rows, acc, *, f, nb, nb_pad, spa, rpw):
    wid = lax.axis_index("s") * SC_CORES + lax.axis_index("c")
    pltpu.sync_copy(lp_hbm, lp_v)
    ninf = jnp.full((16,), -jnp.inf, jnp.float32)
    nsl = f // 16
    acc_rows = rpw * BUCKET_NODES
    dump = acc_rows
    jlo = wid * rpw
    base_node = jlo * BUCKET_NODES

    def init_row(i, _):
        for k in range(nsl):
            acc[i, pl.ds(k * 16, 16)] = ninf
        return 0

    lax.fori_loop(0, acc_rows + 1, init_row, 0)

    def per_worker(w2, _):
        lo = _ext_i32(lp_v, w2 * nb_pad + jnp.minimum(jlo, nb_pad - 1))
        hi = _ext_i32(lp_v, w2 * nb_pad + jnp.minimum(jlo + rpw, nb_pad - 1))
        span = jnp.clip(hi - lo, 0, spa)
        lo = jnp.clip(lo, 0, spa - SCAT_CHUNK)
        nch = (span + SCAT_CHUNK - 1) >> 6

        def chunk(t, _):
            off = pl.multiple_of(
                w2 * spa + jnp.minimum(lo + t * SCAT_CHUNK, spa - SCAT_CHUNK), 8)
            pltpu.sync_copy(dsts_hbm.at[pl.ds(off, SCAT_CHUNK)], ddv)
            pltpu.sync_copy(msg_hbm.at[pl.ds(off, SCAT_CHUNK)], rows)

            for g in range(SCAT_CHUNK // 16):
                ldg = jnp.clip(ddv[pl.ds(g * 16, 16)] - base_node,
                               0, acc_rows - 1)
                for k in range(16):
                    i = g * 16 + k
                    ok = (t * SCAT_CHUNK + i) < span
                    ld = jnp.where(ok, ldg[k], dump)
                    for s in range(nsl):
                        sl = pl.ds(s * 16, 16)
                        acc[ld, sl] = jnp.maximum(acc[ld, sl], rows[i, sl])
            return 0

        lax.fori_loop(0, nch, chunk, 0)
        return 0

    lax.fori_loop(0, SC_WORKERS, per_worker, 0)
    pltpu.sync_copy(acc.at[pl.ds(0, acc_rows)],
                    agg_hbm.at[pl.ds(base_node, acc_rows)])


def _sc_segment_max(msg, staged, n):
    eas, dsts, srcs, cnt, lp = staged
    tot, f = msg.shape
    nb = (n + BUCKET_NODES - 1) // BUCKET_NODES
    nb_pad = ((nb + 15) // 16) * 16
    spa = tot // SC_WORKERS
    rpw = (nb + SC_WORKERS - 1) // SC_WORKERS
    kfn = pl.kernel(
        functools.partial(_scatmax_body, f=f, nb=nb, nb_pad=nb_pad,
                          spa=spa, rpw=rpw),
        out_type=jax.ShapeDtypeStruct((SC_WORKERS * rpw * BUCKET_NODES, f),
                                      jnp.float32),
        mesh=_sc_mesh(),
        compiler_params=pltpu.CompilerParams(use_tc_tiling_on_sc=False,
                                             needs_layout_passes=False),
        scratch_types=[
            pltpu.VMEM((SC_WORKERS * nb_pad,), jnp.int32),
            pltpu.VMEM((SCAT_CHUNK,), jnp.int32),
            pltpu.VMEM((SCAT_CHUNK, f), jnp.float32),
            pltpu.VMEM((rpw * BUCKET_NODES + 8, f), jnp.float32),
        ],
    )
    aggp = kfn(msg, dsts, lp)
    return aggp[:n]


def _update_body(x_ref, agg_ref, uxT_ref, uaT_ref, ub1_ref, uw2T_ref, ub2_ref, out_ref):
    agg = agg_ref[...]
    agg = jnp.where(jnp.isfinite(agg), agg, 0.0)
    hidden = (jnp.dot(x_ref[...], uxT_ref[...], preferred_element_type=jnp.float32)
              + jnp.dot(agg, uaT_ref[...], preferred_element_type=jnp.float32)
              + ub1_ref[...])
    hidden = _lrelu(hidden)
    h = jnp.dot(hidden, uw2T_ref[...], preferred_element_type=jnp.float32) + ub2_ref[...]
    out_ref[...] = jnp.maximum(h, 0.0)


def _node_update(x, agg, uxT, uaT, ub1, uw2T, ub2):
    n = x.shape[0]
    fo = uw2T.shape[1]
    return pl.pallas_call(
        _update_body,
        out_shape=jax.ShapeDtypeStruct((n, fo), jnp.float32),
    )(x, agg, uxT, uaT, ub1, uw2T, ub2)


def _final_body(h_ref, fwT_ref, fb_ref, out_ref):
    out_ref[...] = jnp.dot(h_ref[...], fwT_ref[...], preferred_element_type=jnp.float32) + fb_ref[...]


def _final(h, fwT, fb):
    n = h.shape[0]
    fo = fwT.shape[1]
    return pl.pallas_call(
        _final_body,
        out_shape=jax.ShapeDtypeStruct((n, fo), jnp.float32),
    )(h, fwT, fb)


def _mp_layer(x, staged, mW1, mb1, mW2, mb2, uW1, ub1, uW2, ub2):
    f = x.shape[1]
    wdT = jnp.transpose(mW1[:, :f])
    wsT = jnp.transpose(mW1[:, f:2 * f])
    weT = jnp.transpose(mW1[:, 2 * f:])
    xa, xb = _node_pre(x, wdT, wsT)
    g = _sc_gather_add(xa, xb, staged)
    msg = _edge_dense(g, staged[0], weT, mb1[None, :], jnp.transpose(mW2), mb2[None, :])
    agg = _sc_segment_max(msg, staged, x.shape[0])
    uxT = jnp.transpose(uW1[:, :f])
    uaT = jnp.transpose(uW1[:, f:])
    return _node_update(x, agg, uxT, uaT, ub1[None, :], jnp.transpose(uW2), ub2[None, :])


def kernel(feature, edge_index, edge_attr, mp1_mW1, mp1_mb1, mp1_mW2, mp1_mb2,
           mp1_uW1, mp1_ub1, mp1_uW2, mp1_ub2, mp2_mW1, mp2_mb1, mp2_mW2, mp2_mb2,
           mp2_uW1, mp2_ub1, mp2_uW2, mp2_ub2, fW, fb):
    src = edge_index[0]
    dst = edge_index[1]
    staged = _sc_bucket(dst, src, edge_attr, feature.shape[0])
    h = _mp_layer(feature, staged,
                  mp1_mW1, mp1_mb1, mp1_mW2, mp1_mb2,
                  mp1_uW1, mp1_ub1, mp1_uW2, mp1_ub2)
    h = _mp_layer(h, staged,
                  mp2_mW1, mp2_mb1, mp2_mW2, mp2_mb2,
                  mp2_uW1, mp2_ub1, mp2_uW2, mp2_ub2)
    return _final(h, jnp.transpose(fW), fb[None, :])
